# baseline (speedup 1.0000x reference)
from functools import partial

import jax
import jax.numpy as jnp
from jax.experimental import pallas as pl
from jax.experimental.pallas import tpu as pltpu

POOL_H, POOL_W = 7, 7


def _roi_kernel(meta_ref, fm_ref, out_ref, t_ref, ping_ref, pong_ref, *, n_rois, hw, grp):
    b = pl.program_id(0)
    gi = pl.program_id(1)
    n = hw * hw

    @pl.when(gi == 0)
    def _build():
        def shmax(dst, doff, src, soff, shift):
            dst[doff : doff + n - shift] = jnp.maximum(
                src[soff : soff + n - shift], src[soff + shift : soff + n]
            )

        shmax(ping_ref, 0, fm_ref, 0, 1)
        shmax(pong_ref, 0, ping_ref, 0, 2)
        for lvl, src in ((0, ping_ref), (3, pong_ref), (6, ping_ref)):
            if lvl == 6:
                shmax(ping_ref, 0, pong_ref, 0, 4)
            shmax(t_ref, (lvl + 0) * n, src, 0, hw)
            shmax(t_ref, (lvl + 1) * n, t_ref, (lvl + 0) * n, 2 * hw)
            shmax(t_ref, (lvl + 2) * n, t_ref, (lvl + 1) * n, 4 * hw)

    for rr in range(grp):
        base = (b * n_rois + gi * grp + rr) * 32
        ha = [meta_ref[base + i] for i in range(POOL_H)]
        hb = [meta_ref[base + 7 + i] for i in range(POOL_H)]
        wa = [meta_ref[base + 14 + j] for j in range(POOL_W)]
        wb = [meta_ref[base + 21 + j] for j in range(POOL_W)]
        for i in range(POOL_H):
            for j in range(POOL_W):
                v = jnp.maximum(
                    jnp.maximum(t_ref[ha[i] + wa[j], 0], t_ref[ha[i] + wb[j], 0]),
                    jnp.maximum(t_ref[hb[i] + wa[j], 0], t_ref[hb[i] + wb[j], 0]),
                )
                out_ref[0, rr, i, j, :] = v


def kernel(feature_map, rois):
    bsz, h, w, c = feature_map.shape
    n_rois = rois.shape[1]
    n = h * w

    h0 = (h * rois[..., 0]).astype(jnp.int32)
    w0 = (w * rois[..., 1]).astype(jnp.int32)
    h1 = (h * rois[..., 2]).astype(jnp.int32)
    w1 = (w * rois[..., 3]).astype(jnp.int32)
    hs = jnp.maximum((h1 - h0) // POOL_H, 0)
    ws = jnp.maximum((w1 - w0) // POOL_W, 0)
    hw_max = h - 1

    def addrs(lo0, hi_end, step, nbins, lane_scale, lvl_scale):
        i = jnp.arange(nbins, dtype=jnp.int32)
        lo = lo0[..., None] + i * step[..., None]
        hi = jnp.where(i == nbins - 1, hi_end[..., None], lo + step[..., None])
        ln = hi - lo
        k = jnp.clip(
            (ln >= 2).astype(jnp.int32)
            + (ln >= 4).astype(jnp.int32)
            + (ln >= 8).astype(jnp.int32),
            1,
            3,
        )
        a = jnp.clip(lo, 0, hw_max) * lane_scale + (k - 1) * lvl_scale
        bq = jnp.clip(hi - (1 << k), 0, hw_max) * lane_scale + (k - 1) * lvl_scale
        return a, bq

    ha, hb = addrs(h0, h1, hs, POOL_H, 1, 3 * n)
    wa, wb = addrs(w0, w1, ws, POOL_W, h, n)
    zero = jnp.zeros(ha.shape[:2] + (4,), jnp.int32)
    meta = jnp.concatenate([ha, hb, wa, wb, zero], axis=-1)
    meta_flat = meta.reshape(-1)

    fm_flat = jnp.swapaxes(feature_map, 1, 2).reshape(bsz * n, 1, c)

    grp = 8
    body = partial(_roi_kernel, n_rois=n_rois, hw=h, grp=grp)
    return pl.pallas_call(
        body,
        out_shape=jax.ShapeDtypeStruct((bsz, n_rois, POOL_H, POOL_W, c), jnp.float32),
        grid_spec=pltpu.PrefetchScalarGridSpec(
            num_scalar_prefetch=1,
            grid=(bsz, n_rois // grp),
            in_specs=[pl.BlockSpec((n, 1, c), lambda b, g, *_: (b, 0, 0))],
            out_specs=pl.BlockSpec(
                (1, grp, POOL_H, POOL_W, c), lambda b, g, *_: (b, g, 0, 0, 0)
            ),
            scratch_shapes=[
                pltpu.VMEM((9 * n, 1, c), jnp.float32),
                pltpu.VMEM((n, 1, c), jnp.float32),
                pltpu.VMEM((n, 1, c), jnp.float32),
            ],
        ),
        compiler_params=pltpu.CompilerParams(
            dimension_semantics=("parallel", "arbitrary"),
            vmem_limit_bytes=56 * 1024 * 1024,
        ),
        name="roi_pool",
    )(meta_flat, fm_flat)


# speedup vs baseline: 1.1517x; 1.1517x over previous
from functools import partial

import jax
import jax.numpy as jnp
from jax.experimental import pallas as pl
from jax.experimental.pallas import tpu as pltpu

POOL_H, POOL_W = 7, 7


def _roi_kernel(meta_ref, fm_ref, out_ref, t_ref, ping_ref, pong_ref, *, n_rois, hw, grp):
    b = pl.program_id(0)
    gi = pl.program_id(1)
    n = hw * hw

    @pl.when(gi == 0)
    def _build():
        def shmax(dst, doff, src, soff, shift):
            dst[doff : doff + n - shift] = jnp.maximum(
                src[soff : soff + n - shift], src[soff + shift : soff + n]
            )

        shmax(ping_ref, 0, fm_ref, 0, hw)
        shmax(pong_ref, 0, ping_ref, 0, 2 * hw)
        for lvl, src in ((0, ping_ref), (3, pong_ref), (6, ping_ref)):
            if lvl == 6:
                shmax(ping_ref, 0, pong_ref, 0, 4 * hw)
            shmax(t_ref, (lvl + 0) * n, src, 0, 1)
            shmax(t_ref, (lvl + 1) * n, t_ref, (lvl + 0) * n, 2)
            shmax(t_ref, (lvl + 2) * n, t_ref, (lvl + 1) * n, 4)

    for rr in range(grp):
        base = (b * n_rois + gi * grp + rr) * 32
        ha = [meta_ref[base + i] for i in range(POOL_H)]
        hb = [meta_ref[base + 7 + i] for i in range(POOL_H)]
        wa = [meta_ref[base + 14 + j] for j in range(POOL_W)]
        wb = [meta_ref[base + 21 + j] for j in range(POOL_W)]
        for i in range(POOL_H):
            for j in range(POOL_W):
                v = jnp.maximum(
                    jnp.maximum(t_ref[ha[i] + wa[j], 0], t_ref[ha[i] + wb[j], 0]),
                    jnp.maximum(t_ref[hb[i] + wa[j], 0], t_ref[hb[i] + wb[j], 0]),
                )
                out_ref[0, rr, i, j, :] = v


def kernel(feature_map, rois):
    bsz, h, w, c = feature_map.shape
    n_rois = rois.shape[1]
    n = h * w

    h0 = (h * rois[..., 0]).astype(jnp.int32)
    w0 = (w * rois[..., 1]).astype(jnp.int32)
    h1 = (h * rois[..., 2]).astype(jnp.int32)
    w1 = (w * rois[..., 3]).astype(jnp.int32)
    hs = jnp.maximum((h1 - h0) // POOL_H, 0)
    ws = jnp.maximum((w1 - w0) // POOL_W, 0)
    hw_max = h - 1

    def addrs(lo0, hi_end, step, nbins, lane_scale, lvl_scale):
        i = jnp.arange(nbins, dtype=jnp.int32)
        lo = lo0[..., None] + i * step[..., None]
        hi = jnp.where(i == nbins - 1, hi_end[..., None], lo + step[..., None])
        ln = hi - lo
        k = jnp.clip(
            (ln >= 2).astype(jnp.int32)
            + (ln >= 4).astype(jnp.int32)
            + (ln >= 8).astype(jnp.int32),
            1,
            3,
        )
        a = jnp.clip(lo, 0, hw_max) * lane_scale + (k - 1) * lvl_scale
        bq = jnp.clip(hi - (1 << k), 0, hw_max) * lane_scale + (k - 1) * lvl_scale
        return a, bq

    ha, hb = addrs(h0, h1, hs, POOL_H, w, 3 * n)
    wa, wb = addrs(w0, w1, ws, POOL_W, 1, n)
    zero = jnp.zeros(ha.shape[:2] + (4,), jnp.int32)
    meta = jnp.concatenate([ha, hb, wa, wb, zero], axis=-1)
    meta_flat = meta.reshape(-1)

    fm_flat = feature_map.reshape(bsz * n, 1, c)

    grp = 16
    body = partial(_roi_kernel, n_rois=n_rois, hw=h, grp=grp)
    return pl.pallas_call(
        body,
        out_shape=jax.ShapeDtypeStruct((bsz, n_rois, POOL_H, POOL_W, c), jnp.float32),
        grid_spec=pltpu.PrefetchScalarGridSpec(
            num_scalar_prefetch=1,
            grid=(bsz, n_rois // grp),
            in_specs=[pl.BlockSpec((n, 1, c), lambda b, g, *_: (b, 0, 0))],
            out_specs=pl.BlockSpec(
                (1, grp, POOL_H, POOL_W, c), lambda b, g, *_: (b, g, 0, 0, 0)
            ),
            scratch_shapes=[
                pltpu.VMEM((9 * n, 1, c), jnp.float32),
                pltpu.VMEM((n, 1, c), jnp.float32),
                pltpu.VMEM((n, 1, c), jnp.float32),
            ],
        ),
        compiler_params=pltpu.CompilerParams(
            dimension_semantics=("parallel", "arbitrary"),
            vmem_limit_bytes=56 * 1024 * 1024,
        ),
        name="roi_pool",
    )(meta_flat, fm_flat)
